# baseline (device time: 195644 ns/iter reference)
import jax
import jax.numpy as jnp
from jax import lax
from jax.experimental import pallas as pl
from jax.experimental.pallas import tpu as pltpu

N_DEV = 4


def kernel(x, w_mat):
    k_tot, k_blk = x.shape
    _, n_tot = w_mat.shape
    m_blk = k_tot // N_DEV
    n_tile = 1024
    n_tiles = n_tot // n_tile

    def body(x_hbm, w_hbm, out_hbm,
             xstage_ref, send_ref, comm_ref, y_ref, wbuf_ref, wbf_ref,
             ostage_ref, amax_src_ref, amax_comm_ref,
             send_sems, recv_sems, amax_send_sems, amax_recv_sems,
             xl_sems, w_sems, out_sems):
        me = lax.axis_index("i")

        barrier = pltpu.get_barrier_semaphore()
        for t in range(1, N_DEV):
            peer = lax.rem(me + t, N_DEV)
            pl.semaphore_signal(barrier, inc=1, device_id=(peer,),
                                device_id_type=pl.DeviceIdType.MESH)
        pl.semaphore_wait(barrier, N_DEV - 1)

        def xload(t, buf):
            rows = lax.rem(me + t, N_DEV) * m_blk
            return pltpu.make_async_copy(
                x_hbm.at[pl.ds(rows, m_blk), :], xstage_ref.at[buf],
                xl_sems.at[buf])

        def make_rdma(t):
            target = lax.rem(me + t, N_DEV)
            slot = N_DEV - t
            return pltpu.make_async_remote_copy(
                src_ref=send_ref.at[t],
                dst_ref=comm_ref.at[slot],
                send_sem=send_sems.at[t],
                recv_sem=recv_sems.at[slot],
                device_id=(target,),
                device_id_type=pl.DeviceIdType.MESH,
            )

        cp1 = xload(1, 0)
        cp1.start()
        cp3 = xload(3, 1)
        cp3.start()

        cp1.wait()
        send_ref[1] = xstage_ref[0].astype(jnp.bfloat16)
        cp0 = xload(0, 0)
        cp0.start()
        rdma1 = make_rdma(1)
        rdma1.start()

        cp3.wait()
        send_ref[3] = xstage_ref[1].astype(jnp.bfloat16)
        cp2 = xload(2, 1)
        cp2.start()
        rdma3 = make_rdma(3)
        rdma3.start()

        cp0.wait()
        send_ref[0] = xstage_ref[0].astype(jnp.bfloat16)

        def make_wcopy(k_idx, n, buf):
            return pltpu.make_async_copy(
                w_hbm.at[pl.ds(k_idx * k_blk, k_blk),
                         pl.ds(n * n_tile, n_tile)],
                wbuf_ref.at[buf], w_sems.at[buf])

        def sweep(slot, first, last, amax_in):
            k_idx = lax.rem(me + slot, N_DEV)
            xsrc = send_ref if slot == 0 else comm_ref
            make_wcopy(k_idx, 0, 0).start()
            make_wcopy(k_idx, 0, 0).wait()
            wbf_ref[0] = wbuf_ref[0].astype(jnp.bfloat16)
            make_wcopy(k_idx, 1, 1).start()

            amax = amax_in
            for n in range(n_tiles):
                cur = n % 2
                acc = lax.dot_general(
                    xsrc[slot], wbf_ref[cur],
                    (((1,), (0,)), ((), ())),
                    preferred_element_type=jnp.float32)

                if n + 1 < n_tiles:
                    make_wcopy(k_idx, n + 1, 1 - cur).wait()
                    wbf_ref[1 - cur] = wbuf_ref[1 - cur].astype(jnp.bfloat16)
                if n + 2 < n_tiles:
                    make_wcopy(k_idx, n + 2, cur).start()

                nds = pl.ds(n * n_tile, n_tile)
                if not first:
                    acc = acc + y_ref[:, nds].astype(jnp.float32)
                if last:
                    acc = jnp.maximum(acc, 0.0)
                    amax = jnp.maximum(amax, jnp.max(acc))
                y_ref[:, nds] = acc.astype(jnp.bfloat16)
            return amax

        sweep(0, True, False, jnp.float32(0.0))

        cp2.wait()
        send_ref[2] = xstage_ref[1].astype(jnp.bfloat16)
        rdma2 = make_rdma(2)
        rdma2.start()

        rdma1.wait_recv()
        sweep(3, False, False, jnp.float32(0.0))
        rdma3.wait_recv()
        sweep(1, False, False, jnp.float32(0.0))
        rdma2.wait_recv()
        amax = sweep(2, False, True, jnp.float32(0.0))

        rdma1.wait_send()
        rdma3.wait_send()
        rdma2.wait_send()

        amax_src_ref[...] = jnp.full((8, 128), amax, jnp.float32)
        amax_comm_ref[0] = amax_src_ref[...]
        a_rdmas = []
        for t in range(1, N_DEV):
            target = lax.rem(me + t, N_DEV)
            slot = N_DEV - t
            r = pltpu.make_async_remote_copy(
                src_ref=amax_src_ref,
                dst_ref=amax_comm_ref.at[slot],
                send_sem=amax_send_sems.at[slot],
                recv_sem=amax_recv_sems.at[slot],
                device_id=(target,),
                device_id_type=pl.DeviceIdType.MESH,
            )
            r.start()
            a_rdmas.append(r)
        for r in a_rdmas:
            r.wait()

        g_amax = jnp.max(amax_comm_ref[...])
        inv = 127.0 / g_amax
        scale = g_amax / 127.0

        def make_ocopy(n, buf):
            return pltpu.make_async_copy(
                ostage_ref.at[buf], out_hbm.at[:, pl.ds(n * n_tile, n_tile)],
                out_sems.at[buf])

        for n in range(n_tiles):
            buf = n % 2
            if n >= 2:
                make_ocopy(n - 2, buf).wait()
            yt = y_ref[:, pl.ds(n * n_tile, n_tile)].astype(jnp.float32)
            q = jnp.clip(jnp.round(yt * inv), -127.0, 127.0)
            ostage_ref[buf] = (q * scale).astype(jnp.bfloat16)
            make_ocopy(n, buf).start()

        make_ocopy(n_tiles - 2, (n_tiles - 2) % 2).wait()
        make_ocopy(n_tiles - 1, (n_tiles - 1) % 2).wait()

    return pl.pallas_call(
        body,
        out_shape=jax.ShapeDtypeStruct((m_blk, n_tot), jnp.bfloat16),
        in_specs=[pl.BlockSpec(memory_space=pl.ANY),
                  pl.BlockSpec(memory_space=pl.ANY)],
        out_specs=pl.BlockSpec(memory_space=pl.ANY),
        scratch_shapes=[
            pltpu.VMEM((2, m_blk, k_blk), jnp.float32),
            pltpu.VMEM((N_DEV, m_blk, k_blk), jnp.bfloat16),
            pltpu.VMEM((N_DEV, m_blk, k_blk), jnp.bfloat16),
            pltpu.VMEM((m_blk, n_tot), jnp.bfloat16),
            pltpu.VMEM((2, k_blk, n_tile), jnp.float32),
            pltpu.VMEM((2, k_blk, n_tile), jnp.bfloat16),
            pltpu.VMEM((2, m_blk, n_tile), jnp.bfloat16),
            pltpu.VMEM((8, 128), jnp.float32),
            pltpu.VMEM((N_DEV, 8, 128), jnp.float32),
            pltpu.SemaphoreType.DMA((N_DEV,)),
            pltpu.SemaphoreType.DMA((N_DEV,)),
            pltpu.SemaphoreType.DMA((N_DEV,)),
            pltpu.SemaphoreType.DMA((N_DEV,)),
            pltpu.SemaphoreType.DMA((2,)),
            pltpu.SemaphoreType.DMA((2,)),
            pltpu.SemaphoreType.DMA((2,)),
        ],
        compiler_params=pltpu.CompilerParams(
            collective_id=0, vmem_limit_bytes=64 * 1024 * 1024),
    )(x, w_mat)


# device time: 126937 ns/iter; 1.5413x vs baseline; 1.5413x over previous
import jax
import jax.numpy as jnp
from jax import lax
from jax.experimental import pallas as pl
from jax.experimental.pallas import tpu as pltpu

N_DEV = 4


def kernel(x, w_mat):
    k_tot, k_blk = x.shape
    _, n_tot = w_mat.shape
    m_blk = k_tot // N_DEV
    n_tile = 1024
    n_tiles = n_tot // n_tile

    def body(x_hbm, w_hbm, out_hbm,
             xstage_ref, send_ref, comm_ref, y_ref, wbuf_ref,
             ostage_ref, amax_src_ref, amax_comm_ref,
             send_sems, recv_sems, amax_send_sems, amax_recv_sems,
             xl_sems, w_sems, out_sems):
        me = lax.axis_index("i")

        scope_entry = jax.named_scope("entry")
        scope_entry.__enter__()
        barrier = pltpu.get_barrier_semaphore()
        for t in range(1, N_DEV):
            peer = lax.rem(me + t, N_DEV)
            pl.semaphore_signal(barrier, inc=1, device_id=(peer,),
                                device_id_type=pl.DeviceIdType.MESH)
        pl.semaphore_wait(barrier, N_DEV - 1)

        def xload(t, buf):
            rows = lax.rem(me + t, N_DEV) * m_blk
            return pltpu.make_async_copy(
                x_hbm.at[pl.ds(rows, m_blk), :], xstage_ref.at[buf],
                xl_sems.at[buf])

        def make_rdma(t):
            target = lax.rem(me + t, N_DEV)
            slot = N_DEV - t
            return pltpu.make_async_remote_copy(
                src_ref=send_ref.at[t],
                dst_ref=comm_ref.at[slot],
                send_sem=send_sems.at[t],
                recv_sem=recv_sems.at[slot],
                device_id=(target,),
                device_id_type=pl.DeviceIdType.MESH,
            )

        cp1 = xload(1, 0)
        cp1.start()
        cp3 = xload(3, 1)
        cp3.start()

        cp1.wait()
        send_ref[1] = xstage_ref[0].astype(jnp.bfloat16)
        cp0 = xload(0, 0)
        cp0.start()
        rdma1 = make_rdma(1)
        rdma1.start()

        cp3.wait()
        send_ref[3] = xstage_ref[1].astype(jnp.bfloat16)
        cp2 = xload(2, 1)
        cp2.start()
        rdma3 = make_rdma(3)
        rdma3.start()

        cp0.wait()
        send_ref[0] = xstage_ref[0].astype(jnp.bfloat16)
        scope_entry.__exit__(None, None, None)

        def make_wcopy(k_idx, n, buf):
            return pltpu.make_async_copy(
                w_hbm.at[pl.ds(k_idx * k_blk, k_blk),
                         pl.ds(n * n_tile, n_tile)],
                wbuf_ref.at[buf], w_sems.at[buf])

        def sweep(slot, first, last, amax_in):
            k_idx = lax.rem(me + slot, N_DEV)
            xsrc = send_ref if slot == 0 else comm_ref
            make_wcopy(k_idx, 0, 0).start()

            def step(n, amax):
                buf = lax.rem(n, 2)
                make_wcopy(k_idx, n, buf).wait()

                @pl.when(n + 1 < n_tiles)
                def _():
                    make_wcopy(k_idx, n + 1, 1 - buf).start()

                acc = lax.dot_general(
                    xsrc[slot], wbuf_ref[buf].astype(jnp.bfloat16),
                    (((1,), (0,)), ((), ())),
                    preferred_element_type=jnp.float32)
                nds = pl.ds(n * n_tile, n_tile)
                if not first:
                    acc = acc + y_ref[:, nds].astype(jnp.float32)
                if last:
                    acc = jnp.maximum(acc, 0.0)
                    amax = jnp.maximum(amax, jnp.max(acc))
                y_ref[:, nds] = acc.astype(jnp.bfloat16)
                return amax

            return lax.fori_loop(0, n_tiles, step, amax_in)

        with jax.named_scope("sweep0"):
            sweep(0, True, False, jnp.float32(0.0))

        with jax.named_scope("diag_send"):
            cp2.wait()
            send_ref[2] = xstage_ref[1].astype(jnp.bfloat16)
            rdma2 = make_rdma(2)
            rdma2.start()

        with jax.named_scope("wait_slot3"):
            rdma1.wait_recv()
        with jax.named_scope("sweep3"):
            sweep(3, False, False, jnp.float32(0.0))
        with jax.named_scope("wait_slot1"):
            rdma3.wait_recv()
        with jax.named_scope("sweep1"):
            sweep(1, False, False, jnp.float32(0.0))
        with jax.named_scope("wait_slot2"):
            rdma2.wait_recv()
        with jax.named_scope("sweep2"):
            amax = sweep(2, False, True, jnp.float32(0.0))

        with jax.named_scope("amax_xchg"):
            rdma1.wait_send()
            rdma3.wait_send()
            rdma2.wait_send()

            amax_src_ref[...] = jnp.full((8, 128), amax, jnp.float32)
            amax_comm_ref[0] = amax_src_ref[...]
            a_rdmas = []
            for t in range(1, N_DEV):
                target = lax.rem(me + t, N_DEV)
                slot = N_DEV - t
                r = pltpu.make_async_remote_copy(
                    src_ref=amax_src_ref,
                    dst_ref=amax_comm_ref.at[slot],
                    send_sem=amax_send_sems.at[slot],
                    recv_sem=amax_recv_sems.at[slot],
                    device_id=(target,),
                    device_id_type=pl.DeviceIdType.MESH,
                )
                r.start()
                a_rdmas.append(r)
            for r in a_rdmas:
                r.wait()

        g_amax = jnp.max(amax_comm_ref[...])
        inv = 127.0 / g_amax
        scale = g_amax / 127.0

        def make_ocopy(n, buf):
            return pltpu.make_async_copy(
                ostage_ref.at[buf], out_hbm.at[:, pl.ds(n * n_tile, n_tile)],
                out_sems.at[buf])

        with jax.named_scope("quant"):
            for n in range(n_tiles):
                buf = n % 2
                if n >= 2:
                    make_ocopy(n - 2, buf).wait()
                yt = y_ref[:, pl.ds(n * n_tile, n_tile)].astype(jnp.float32)
                q = jnp.clip(jnp.round(yt * inv), -127.0, 127.0)
                ostage_ref[buf] = (q * scale).astype(jnp.bfloat16)
                make_ocopy(n, buf).start()

            make_ocopy(n_tiles - 2, (n_tiles - 2) % 2).wait()
            make_ocopy(n_tiles - 1, (n_tiles - 1) % 2).wait()

    return pl.pallas_call(
        body,
        out_shape=jax.ShapeDtypeStruct((m_blk, n_tot), jnp.bfloat16),
        in_specs=[pl.BlockSpec(memory_space=pl.ANY),
                  pl.BlockSpec(memory_space=pl.ANY)],
        out_specs=pl.BlockSpec(memory_space=pl.ANY),
        scratch_shapes=[
            pltpu.VMEM((2, m_blk, k_blk), jnp.float32),
            pltpu.VMEM((N_DEV, m_blk, k_blk), jnp.bfloat16),
            pltpu.VMEM((N_DEV, m_blk, k_blk), jnp.bfloat16),
            pltpu.VMEM((m_blk, n_tot), jnp.bfloat16),
            pltpu.VMEM((2, k_blk, n_tile), jnp.float32),
            pltpu.VMEM((2, m_blk, n_tile), jnp.bfloat16),
            pltpu.VMEM((8, 128), jnp.float32),
            pltpu.VMEM((N_DEV, 8, 128), jnp.float32),
            pltpu.SemaphoreType.DMA((N_DEV,)),
            pltpu.SemaphoreType.DMA((N_DEV,)),
            pltpu.SemaphoreType.DMA((N_DEV,)),
            pltpu.SemaphoreType.DMA((N_DEV,)),
            pltpu.SemaphoreType.DMA((2,)),
            pltpu.SemaphoreType.DMA((2,)),
            pltpu.SemaphoreType.DMA((2,)),
        ],
        compiler_params=pltpu.CompilerParams(
            collective_id=0, vmem_limit_bytes=64 * 1024 * 1024),
    )(x, w_mat)
